# baseline (speedup 1.0000x reference)
import jax
import jax.numpy as jnp
from jax.experimental import pallas as pl


def _layer_norm(x, g, b, eps=1e-5):
    m = jnp.mean(x, axis=-1, keepdims=True)
    v = jnp.mean((x - m) ** 2, axis=-1, keepdims=True)
    return g * (x - m) / jnp.sqrt(v + eps) + b


def _gcn_conv(x, edge_index, W, b):
    M = x.shape[0]
    loop = jnp.arange(M, dtype=edge_index.dtype)
    src = jnp.concatenate([edge_index[0], loop])
    dst = jnp.concatenate([edge_index[1], loop])
    h = x @ W
    deg = jnp.zeros((M,), x.dtype).at[dst].add(1.0)
    dinv = jax.lax.rsqrt(deg)
    norm = dinv[src] * dinv[dst]
    out = jnp.zeros_like(h).at[dst].add(h[src] * norm[:, None])
    return out + b


def _seg_mean(x, ids, num):
    s = jax.ops.segment_sum(x, ids, num_segments=num)
    c = jax.ops.segment_sum(jnp.ones((x.shape[0],), x.dtype), ids, num_segments=num)
    return s / jnp.maximum(c, 1.0)[:, None]


def _mlp_body(gx_ref, w1_ref, b1_ref, w2_ref, b2_ref, out_ref):
    h = jnp.maximum(gx_ref[...] @ w1_ref[...] + b1_ref[...], 0.0)
    out_ref[...] = h @ w2_ref[...] + b2_ref[...]


def kernel(x, eg_orig_node_idx, eg_node_to_hedge, eg_edge_index, ng_orig_edge_idx, ng_hedge_to_node, ng_edge_index, node_to_graph, W_n2e_0, b_n2e_0, W_e2n_0, b_e2n_0, W_n2e_1, b_n2e_1, W_e2n_1, b_e2n_1, ln_e_g, ln_e_b, ln_n_g, ln_n_b, lin1_W, lin1_b, lin2_W, lin2_b):
    H = 25000
    N = x.shape[0]
    G = 64

    def layer(nx, Wn, bn, We, be):
        _nx = _gcn_conv(nx[eg_orig_node_idx], eg_edge_index, Wn, bn)
        ex = _seg_mean(_nx, eg_node_to_hedge, H)
        ex = _layer_norm(ex, ln_e_g, ln_e_b)
        ex = jax.nn.elu(ex)
        _ex = _gcn_conv(ex[ng_orig_edge_idx], ng_edge_index, We, be)
        nx2 = _seg_mean(_ex, ng_hedge_to_node, N)
        nx2 = _layer_norm(nx2, ln_n_g, ln_n_b)
        return jax.nn.elu(nx2)

    nx1 = layer(x, W_n2e_0, b_n2e_0, W_e2n_0, b_e2n_0)
    nx2 = layer(nx1, W_n2e_1, b_n2e_1, W_e2n_1, b_e2n_1)
    node_xs = jnp.concatenate([nx1, nx2], axis=1)
    gx = _seg_mean(node_xs, node_to_graph, G)

    C = lin2_W.shape[1]
    w2p = jnp.zeros((lin2_W.shape[0], 128), jnp.float32).at[:, :C].set(lin2_W)
    b2p = jnp.zeros((128,), jnp.float32).at[:C].set(lin2_b)
    logits = pl.pallas_call(
        _mlp_body,
        out_shape=jax.ShapeDtypeStruct((G, 128), jnp.float32),
    )(gx, lin1_W, lin1_b, w2p, b2p)[:, :C]
    return jax.nn.log_softmax(logits, axis=-1)


# speedup vs baseline: 1.0345x; 1.0345x over previous
import functools

import jax
import jax.numpy as jnp
from jax import lax
from jax.experimental import pallas as pl
from jax.experimental.pallas import tpu as pltpu
from jax.experimental.pallas import tpu_sc as plsc

_NC = 2
_NS = 16
_NW = _NC * _NS
_CH = 128
_D = 128


def _pad_len(n, mult):
    return ((n + mult - 1) // mult) * mult


@functools.lru_cache(maxsize=None)
def _make_gather(M, Kp):
    per_w = Kp // _NW
    n_chunks = per_w // _CH
    mesh = plsc.VectorSubcoreMesh(core_axis_name="c", subcore_axis_name="s")

    @functools.partial(
        pl.kernel,
        mesh=mesh,
        out_type=jax.ShapeDtypeStruct((Kp, _D), jnp.float32),
        scratch_types=[
            pltpu.VMEM((_CH,), jnp.int32),
            pltpu.VMEM((_CH, _D), jnp.float32),
            pltpu.SemaphoreType.DMA,
        ],
    )
    def gk(table_hbm, idx_hbm, out_hbm, idx_v, rows_v, sem):
        wid = lax.axis_index("s") * _NC + lax.axis_index("c")
        base = wid * per_w

        def step(c, carry):
            off = base + c * _CH
            pltpu.sync_copy(idx_hbm.at[pl.ds(off, _CH)], idx_v)
            pltpu.async_copy(table_hbm.at[idx_v], rows_v, sem).wait()
            pltpu.sync_copy(rows_v, out_hbm.at[pl.ds(off, _CH)])
            return carry

        lax.fori_loop(0, n_chunks, step, 0)

    return gk


def _sc_gather(table, idx):
    K = idx.shape[0]
    Kp = _pad_len(K, _NW * _CH)
    if Kp != K:
        idx = jnp.concatenate([idx, jnp.zeros((Kp - K,), jnp.int32)])
    out = _make_gather(table.shape[0], Kp)(table, idx)
    return out[:K]


def _layer_norm(x, g, b, eps=1e-5):
    m = jnp.mean(x, axis=-1, keepdims=True)
    v = jnp.mean((x - m) ** 2, axis=-1, keepdims=True)
    return g * (x - m) / jnp.sqrt(v + eps) + b


def _gcn_conv(x, edge_index, W, b):
    M = x.shape[0]
    loop = jnp.arange(M, dtype=edge_index.dtype)
    src = jnp.concatenate([edge_index[0], loop])
    dst = jnp.concatenate([edge_index[1], loop])
    h = x @ W
    deg = jnp.zeros((M,), x.dtype).at[dst].add(1.0)
    dinv = jax.lax.rsqrt(deg)
    norm = dinv[src] * dinv[dst]
    out = jnp.zeros_like(h).at[dst].add(h[src] * norm[:, None])
    return out + b


def _seg_mean(x, ids, num):
    s = jax.ops.segment_sum(x, ids, num_segments=num)
    c = jax.ops.segment_sum(jnp.ones((x.shape[0],), x.dtype), ids, num_segments=num)
    return s / jnp.maximum(c, 1.0)[:, None]


def _mlp_body(gx_ref, w1_ref, b1_ref, w2_ref, b2_ref, out_ref):
    h = jnp.maximum(gx_ref[...] @ w1_ref[...] + b1_ref[...], 0.0)
    out_ref[...] = h @ w2_ref[...] + b2_ref[...]


def kernel(x, eg_orig_node_idx, eg_node_to_hedge, eg_edge_index, ng_orig_edge_idx, ng_hedge_to_node, ng_edge_index, node_to_graph, W_n2e_0, b_n2e_0, W_e2n_0, b_e2n_0, W_n2e_1, b_n2e_1, W_e2n_1, b_e2n_1, ln_e_g, ln_e_b, ln_n_g, ln_n_b, lin1_W, lin1_b, lin2_W, lin2_b):
    H = 25000
    N = x.shape[0]
    G = 64

    def layer(nx, Wn, bn, We, be):
        _nx = _gcn_conv(_sc_gather(nx, eg_orig_node_idx), eg_edge_index, Wn, bn)
        ex = _seg_mean(_nx, eg_node_to_hedge, H)
        ex = _layer_norm(ex, ln_e_g, ln_e_b)
        ex = jax.nn.elu(ex)
        _ex = _gcn_conv(_sc_gather(ex, ng_orig_edge_idx), ng_edge_index, We, be)
        nx2 = _seg_mean(_ex, ng_hedge_to_node, N)
        nx2 = _layer_norm(nx2, ln_n_g, ln_n_b)
        return jax.nn.elu(nx2)

    nx1 = layer(x, W_n2e_0, b_n2e_0, W_e2n_0, b_e2n_0)
    nx2 = layer(nx1, W_n2e_1, b_n2e_1, W_e2n_1, b_e2n_1)
    node_xs = jnp.concatenate([nx1, nx2], axis=1)
    gx = _seg_mean(node_xs, node_to_graph, G)

    C = lin2_W.shape[1]
    w2p = jnp.zeros((lin2_W.shape[0], 128), jnp.float32).at[:, :C].set(lin2_W)
    b2p = jnp.zeros((128,), jnp.float32).at[:C].set(lin2_b)
    logits = pl.pallas_call(
        _mlp_body,
        out_shape=jax.ShapeDtypeStruct((G, 128), jnp.float32),
    )(gx, lin1_W, lin1_b, w2p, b2p)[:, :C]
    return jax.nn.log_softmax(logits, axis=-1)


# speedup vs baseline: 2.5341x; 2.4496x over previous
import functools

import jax
import jax.numpy as jnp
from jax import lax
from jax.experimental import pallas as pl
from jax.experimental.pallas import tpu as pltpu
from jax.experimental.pallas import tpu_sc as plsc

_NC = 2
_NS = 16
_NW = _NC * _NS
_CH = 128
_D = 128

_N = 50000
_H = 25000
_P = 400000
_E = 400000
_G = 64
_C = 10

_P_PAD = 401408
_H_PAD = 25088
_N_PAD = 50176
_EP_PAD = 802816
_R = 12544

_BIG = 1 << 30


def _pad_len(n, mult):
    return ((n + mult - 1) // mult) * mult


def _mesh():
    return plsc.VectorSubcoreMesh(core_axis_name="c", subcore_axis_name="s",
                                  num_cores=_NC, num_subcores=_NS)


@functools.lru_cache(maxsize=None)
def _make_gather(M, Kp):
    per_w = Kp // _NW
    n_chunks = per_w // _CH

    @functools.partial(
        pl.kernel,
        mesh=_mesh(),
        out_type=jax.ShapeDtypeStruct((Kp, _D), jnp.float32),
        scratch_types=[
            pltpu.VMEM((_CH,), jnp.int32),
            pltpu.VMEM((_CH, _D), jnp.float32),
            pltpu.SemaphoreType.DMA,
        ],
    )
    def gk(table_hbm, idx_hbm, out_hbm, idx_v, rows_v, sem):
        wid = lax.axis_index("s") * _NC + lax.axis_index("c")
        base = wid * per_w

        def step(c, carry):
            off = base + c * _CH
            pltpu.sync_copy(idx_hbm.at[pl.ds(off, _CH)], idx_v)
            pltpu.async_copy(table_hbm.at[idx_v], rows_v, sem).wait()
            pltpu.sync_copy(rows_v, out_hbm.at[pl.ds(off, _CH)])
            return carry

        lax.fori_loop(0, n_chunks, step, 0)

    return gk


def _sc_gather(table, idx_padded):
    return _make_gather(table.shape[0], idx_padded.shape[0])(table, idx_padded)


@functools.lru_cache(maxsize=None)
def _make_counts(Kp, Mp):
    chunks = Kp // _CH
    per_sc = chunks // 2
    per_tec = per_sc // _NS
    nz = Mp // 512
    zbound = (nz + _NS - 1) // _NS

    @functools.partial(
        pl.kernel,
        mesh=_mesh(),
        out_type=jax.ShapeDtypeStruct((2 * Mp,), jnp.float32),
        scratch_types=[
            pltpu.VMEM((_CH,), jnp.int32),
            pltpu.VMEM((_CH,), jnp.float32),
            pltpu.VMEM((512,), jnp.float32),
            pltpu.VMEM_SHARED((Mp,), jnp.float32),
        ],
    )
    def ck(idx_hbm, out_hbm, idx_v, ones_v, zbuf, acc):
        cid = lax.axis_index("c")
        sid = lax.axis_index("s")
        for i in range(32):
            zbuf[pl.ds(16 * i, 16)] = jnp.zeros((16,), jnp.float32)
        for i in range(8):
            ones_v[pl.ds(16 * i, 16)] = jnp.full((16,), 1.0, jnp.float32)
        for i in range(zbound):
            zc = sid + _NS * i

            @pl.when(zc < nz)
            def _():
                pltpu.sync_copy(zbuf, acc.at[pl.ds(zc * 512, 512)])

        plsc.subcore_barrier()

        def step(i, carry):
            c = cid * per_sc + sid * per_tec + i
            pltpu.sync_copy(idx_hbm.at[pl.ds(c * _CH, _CH)], idx_v)
            pltpu.sync_copy(ones_v, acc.at[idx_v], add=True)
            return carry

        lax.fori_loop(0, per_tec, step, 0)
        plsc.subcore_barrier()
        for i in range(zbound):
            zc = sid + _NS * i

            @pl.when(zc < nz)
            def _():
                pltpu.sync_copy(acc.at[pl.ds(zc * 512, 512)], zbuf)
                pltpu.sync_copy(zbuf, out_hbm.at[pl.ds(cid * Mp + zc * 512, 512)])

    return ck


def _sc_counts(idx_padded, Mp):
    out = _make_counts(idx_padded.shape[0], Mp)(idx_padded)
    return out.reshape(2, Mp)


@functools.lru_cache(maxsize=None)
def _make_edge_agg(EPp, HSrows, SEGrows, n_pass):
    chunks = EPp // _CH
    per_tec = chunks // _NS
    nzr = (_R + _CH) // _CH
    nwr = _R // _CH
    zbound = (nzr + _NS - 1) // _NS
    wbound = (nwr + _NS - 1) // _NS

    @functools.partial(
        pl.kernel,
        mesh=_mesh(),
        out_type=jax.ShapeDtypeStruct((n_pass * 2 * _R, _D), jnp.float32),
        scratch_types=[
            pltpu.VMEM((_CH,), jnp.int32),
            pltpu.VMEM((_CH,), jnp.int32),
            pltpu.VMEM((_CH,), jnp.float32),
            pltpu.VMEM((_CH,), jnp.int32),
            pltpu.VMEM((_CH,), jnp.int32),
            pltpu.VMEM((_CH, _D), jnp.float32),
            pltpu.VMEM_SHARED((_R + _CH, _D), jnp.float32),
        ],
    )
    def ek(hs_hbm, src_hbm, dst_hbm, seg_hbm, dinv_hbm, out_hbm,
           src_v, dst_v, dinvd_v, segd_v, lidx_v, rows_v, acc):
        cid = lax.axis_index("c")
        sid = lax.axis_index("s")
        for p in range(n_pass):
            base = (2 * p + cid) * _R

            def zrow(j, carry):
                for q in range(8):
                    rows_v[j, pl.ds(16 * q, 16)] = jnp.zeros((16,), jnp.float32)
                return carry

            lax.fori_loop(0, _CH, zrow, 0)
            for i in range(zbound):
                zc = sid + _NS * i

                @pl.when(zc < nzr)
                def _():
                    pltpu.sync_copy(rows_v, acc.at[pl.ds(zc * _CH, _CH)])

            plsc.subcore_barrier()

            def step(i, carry):
                off = (sid * per_tec + i) * _CH
                pltpu.sync_copy(src_hbm.at[pl.ds(off, _CH)], src_v)
                pltpu.sync_copy(dst_hbm.at[pl.ds(off, _CH)], dst_v)
                pltpu.sync_copy(dinv_hbm.at[dst_v], dinvd_v)
                pltpu.sync_copy(seg_hbm.at[dst_v], segd_v)
                pltpu.sync_copy(hs_hbm.at[src_v], rows_v)
                for q in range(8):
                    sl = pl.ds(16 * q, 16)
                    v = segd_v[sl] - base
                    m = (v >= 0) & (v < _R)
                    lidx_v[sl] = jnp.where(m, v, _R)

                def sgrp(t, c2):
                    w16 = dinvd_v[pl.ds(16 * t, 16)]
                    for l in range(16):
                        s = w16[l]
                        j = 16 * t + l
                        for q in range(8):
                            sl = pl.ds(16 * q, 16)
                            rows_v[j, sl] = rows_v[j, sl] * s
                    return c2

                lax.fori_loop(0, _CH // 16, sgrp, 0)
                pltpu.sync_copy(rows_v, acc.at[lidx_v], add=True)
                return carry

            lax.fori_loop(0, per_tec, step, 0)
            plsc.subcore_barrier()
            for i in range(wbound):
                wc = sid + _NS * i

                @pl.when(wc < nwr)
                def _():
                    pltpu.sync_copy(acc.at[pl.ds(wc * _CH, _CH)], rows_v)
                    pltpu.sync_copy(rows_v, out_hbm.at[pl.ds(base + wc * _CH, _CH)])

            plsc.subcore_barrier()

    return ek


def _sc_edge_agg(hs, src, dst, seg, dinv, n_pass):
    k = _make_edge_agg(src.shape[0], hs.shape[0], seg.shape[0], n_pass)
    return k(hs, src, dst, seg, dinv)


def _tc_matmul(g, W, dinv_col):
    Kp = g.shape[0]
    blk = 512

    def body(g_ref, w_ref, d_ref, o_ref):
        o_ref[...] = jnp.dot(g_ref[...], w_ref[...],
                             preferred_element_type=jnp.float32) * d_ref[...]

    return pl.pallas_call(
        body,
        grid=(Kp // blk,),
        in_specs=[
            pl.BlockSpec((blk, _D), lambda i: (i, 0)),
            pl.BlockSpec((_D, _D), lambda i: (0, 0)),
            pl.BlockSpec((blk, 1), lambda i: (i, 0)),
        ],
        out_specs=pl.BlockSpec((blk, _D), lambda i: (i, 0)),
        out_shape=jax.ShapeDtypeStruct((Kp, _D), jnp.float32),
    )(g, W, dinv_col)


def _tc_dinv(deg_parts):
    rows = deg_parts.shape[1] // _D

    def body(d_ref, o_ref):
        d = d_ref[0] + d_ref[1]
        o_ref[...] = lax.rsqrt(jnp.maximum(d, 1.0))

    return pl.pallas_call(
        body,
        out_shape=jax.ShapeDtypeStruct((rows, _D), jnp.float32),
    )(deg_parts.reshape(2, rows, _D))


def _tc_post(ssum, cparts, bias, ln_g, ln_b):
    Mp = ssum.shape[0]
    blk = 512
    c0 = cparts[0].reshape(Mp, 1)
    c1 = cparts[1].reshape(Mp, 1)

    def body(s_ref, c0_ref, c1_ref, b_ref, g_ref, lb_ref, o_ref):
        c = jnp.maximum(c0_ref[...] + c1_ref[...], 1.0)
        ex = s_ref[...] * (1.0 / c) + b_ref[...]
        m = jnp.mean(ex, axis=-1, keepdims=True)
        v = jnp.mean((ex - m) ** 2, axis=-1, keepdims=True)
        y = g_ref[...] * (ex - m) / jnp.sqrt(v + 1e-5) + lb_ref[...]
        o_ref[...] = jnp.where(y > 0, y, jnp.exp(jnp.minimum(y, 0.0)) - 1.0)

    return pl.pallas_call(
        body,
        grid=(Mp // blk,),
        in_specs=[
            pl.BlockSpec((blk, _D), lambda i: (i, 0)),
            pl.BlockSpec((blk, 1), lambda i: (i, 0)),
            pl.BlockSpec((blk, 1), lambda i: (i, 0)),
            pl.BlockSpec((1, _D), lambda i: (0, 0)),
            pl.BlockSpec((1, _D), lambda i: (0, 0)),
            pl.BlockSpec((1, _D), lambda i: (0, 0)),
        ],
        out_specs=pl.BlockSpec((blk, _D), lambda i: (i, 0)),
        out_shape=jax.ShapeDtypeStruct((Mp, _D), jnp.float32),
    )(ssum, c0, c1, bias.reshape(1, _D), ln_g.reshape(1, _D), ln_b.reshape(1, _D))


def _tc_final(nx1, nx2, n2g3, l1a, l1b2, lb1, w2p, b2p):
    blk = 512
    ngrid = _N_PAD // blk

    def body(x1_ref, x2_ref, id_ref, a_ref, b_ref, lb_ref, w2_ref, b2_ref,
             o_ref, acc1, acc2, cnt):
        i = pl.program_id(0)

        @pl.when(i == 0)
        def _():
            acc1[...] = jnp.zeros_like(acc1)
            acc2[...] = jnp.zeros_like(acc2)
            cnt[...] = jnp.zeros_like(cnt)

        ids = id_ref[0, 0, :]
        oh = (lax.broadcasted_iota(jnp.int32, (_G, blk), 0)
              == ids[None, :]).astype(jnp.float32)
        acc1[...] += jnp.dot(oh, x1_ref[...], preferred_element_type=jnp.float32)
        acc2[...] += jnp.dot(oh, x2_ref[...], preferred_element_type=jnp.float32)
        cnt[...] += jnp.sum(oh, axis=1, keepdims=True)

        @pl.when(i == ngrid - 1)
        def _():
            ci = 1.0 / jnp.maximum(cnt[...], 1.0)
            g1 = acc1[...] * ci
            g2 = acc2[...] * ci
            h1 = jnp.maximum(
                jnp.dot(g1, a_ref[...], preferred_element_type=jnp.float32)
                + jnp.dot(g2, b_ref[...], preferred_element_type=jnp.float32)
                + lb_ref[...], 0.0)
            lg = jnp.dot(h1, w2_ref[...],
                         preferred_element_type=jnp.float32) + b2_ref[...]
            col = lax.broadcasted_iota(jnp.int32, (_G, _D), 1)
            lg = jnp.where(col < _C, lg, -1e30)
            m = jnp.max(lg, axis=-1, keepdims=True)
            lse = jnp.log(jnp.sum(jnp.exp(lg - m), axis=-1, keepdims=True)) + m
            o_ref[...] = lg - lse

    return pl.pallas_call(
        body,
        grid=(ngrid,),
        in_specs=[
            pl.BlockSpec((blk, _D), lambda i: (i, 0)),
            pl.BlockSpec((blk, _D), lambda i: (i, 0)),
            pl.BlockSpec((1, 1, blk), lambda i: (i, 0, 0)),
            pl.BlockSpec((_D, _D), lambda i: (0, 0)),
            pl.BlockSpec((_D, _D), lambda i: (0, 0)),
            pl.BlockSpec((1, _D), lambda i: (0, 0)),
            pl.BlockSpec((_D, _D), lambda i: (0, 0)),
            pl.BlockSpec((1, _D), lambda i: (0, 0)),
        ],
        out_specs=pl.BlockSpec((_G, _D), lambda i: (0, 0)),
        out_shape=jax.ShapeDtypeStruct((_G, _D), jnp.float32),
        scratch_shapes=[
            pltpu.VMEM((_G, _D), jnp.float32),
            pltpu.VMEM((_G, _D), jnp.float32),
            pltpu.VMEM((_G, 1), jnp.float32),
        ],
    )(nx1, nx2, n2g3, l1a, l1b2, lb1, w2p, b2p)


def kernel(x, eg_orig_node_idx, eg_node_to_hedge, eg_edge_index, ng_orig_edge_idx, ng_hedge_to_node, ng_edge_index, node_to_graph, W_n2e_0, b_n2e_0, W_e2n_0, b_e2n_0, W_n2e_1, b_n2e_1, W_e2n_1, b_e2n_1, ln_e_g, ln_e_b, ln_n_g, ln_n_b, lin1_W, lin1_b, lin2_W, lin2_b):
    i32 = jnp.int32
    padP = _P_PAD - _P
    padE = _EP_PAD - (_E + _P)
    loop = jnp.arange(_P, dtype=i32)

    eg_oni = jnp.concatenate([eg_orig_node_idx, jnp.zeros((padP,), i32)])
    ng_oei = jnp.concatenate([ng_orig_edge_idx, jnp.zeros((padP,), i32)])
    seg_eg = jnp.concatenate([eg_node_to_hedge, jnp.full((padP,), _BIG, i32)])
    seg_ng = jnp.concatenate([ng_hedge_to_node, jnp.full((padP,), _BIG, i32)])
    cidx_eg = jnp.concatenate([eg_node_to_hedge, jnp.full((padP,), _H_PAD - 8, i32)])
    cidx_ng = jnp.concatenate([ng_hedge_to_node, jnp.full((padP,), _N_PAD - 8, i32)])
    src_eg = jnp.concatenate([eg_edge_index[0], loop, jnp.zeros((padE,), i32)])
    dst_eg = jnp.concatenate([eg_edge_index[1], loop, jnp.full((padE,), _P, i32)])
    src_ng = jnp.concatenate([ng_edge_index[0], loop, jnp.zeros((padE,), i32)])
    dst_ng = jnp.concatenate([ng_edge_index[1], loop, jnp.full((padE,), _P, i32)])
    n2g3 = jnp.concatenate([node_to_graph, jnp.full((_N_PAD - _N,), _G, i32)]
                           ).reshape(_N_PAD // 512, 1, 512)

    deg_eg = _sc_counts(dst_eg, _P_PAD)
    deg_ng = _sc_counts(dst_ng, _P_PAD)
    c_eg = _sc_counts(cidx_eg, _H_PAD)
    c_ng = _sc_counts(cidx_ng, _N_PAD)

    dinv_eg = _tc_dinv(deg_eg).reshape(_P_PAD)
    dinv_ng = _tc_dinv(deg_ng).reshape(_P_PAD)
    dinv_eg_col = dinv_eg.reshape(_P_PAD, 1)
    dinv_ng_col = dinv_ng.reshape(_P_PAD, 1)

    def layer(nx_full, Wn, bn, We, be):
        g1 = _sc_gather(nx_full, eg_oni)
        hs1 = _tc_matmul(g1, Wn, dinv_eg_col)
        s1 = _sc_edge_agg(hs1, src_eg, dst_eg, seg_eg, dinv_eg, 1)
        ex = _tc_post(s1, c_eg, bn, ln_e_g, ln_e_b)
        g2 = _sc_gather(ex, ng_oei)
        hs2 = _tc_matmul(g2, We, dinv_ng_col)
        s2 = _sc_edge_agg(hs2, src_ng, dst_ng, seg_ng, dinv_ng, 2)
        return _tc_post(s2, c_ng, be, ln_n_g, ln_n_b)

    nx1 = layer(x, W_n2e_0, b_n2e_0, W_e2n_0, b_e2n_0)
    nx2 = layer(nx1, W_n2e_1, b_n2e_1, W_e2n_1, b_e2n_1)

    l1a = lin1_W[:_D]
    l1b2 = lin1_W[_D:]
    w2p = jnp.zeros((_D, _D), jnp.float32).at[:, :_C].set(lin2_W)
    b2p = jnp.zeros((1, _D), jnp.float32).at[0, :_C].set(lin2_b)
    out = _tc_final(nx1, nx2, n2g3, l1a, l1b2, lin1_b.reshape(1, _D), w2p, b2p)
    return out[:, :_C]


# speedup vs baseline: 4.5796x; 1.8072x over previous
import functools

import jax
import jax.numpy as jnp
from jax import lax
from jax.experimental import pallas as pl
from jax.experimental.pallas import tpu as pltpu
from jax.experimental.pallas import tpu_sc as plsc

_NC = 2
_NS = 16
_NW = _NC * _NS
_CH = 128
_ECH = 64
_D = 128

_N = 50000
_H = 25000
_P = 400000
_E = 400000
_G = 64
_C = 10

_P_PAD = 401408
_H_PAD = 25088
_N_PAD = 50176
_EP_PAD = 802816
_R = 12544

_BIG = 1 << 30


def _pad_len(n, mult):
    return ((n + mult - 1) // mult) * mult


def _mesh():
    return plsc.VectorSubcoreMesh(core_axis_name="c", subcore_axis_name="s",
                                  num_cores=_NC, num_subcores=_NS)


@functools.lru_cache(maxsize=None)
def _make_gather(M, Kp):
    per_w = Kp // _NW
    n_chunks = per_w // _CH

    @functools.partial(
        pl.kernel,
        mesh=_mesh(),
        out_type=jax.ShapeDtypeStruct((Kp, _D), jnp.float32),
        scratch_types=[
            pltpu.VMEM((_CH,), jnp.int32),
            pltpu.VMEM((_CH,), jnp.int32),
            pltpu.VMEM((_CH, _D), jnp.float32),
            pltpu.VMEM((_CH, _D), jnp.float32),
            pltpu.SemaphoreType.DMA,
            pltpu.SemaphoreType.DMA,
            pltpu.SemaphoreType.DMA,
            pltpu.SemaphoreType.DMA,
            pltpu.SemaphoreType.DMA,
            pltpu.SemaphoreType.DMA,
        ],
    )
    def gk(table_hbm, idx_hbm, out_hbm, i0, i1, r0, r1, a0, a1, g0, g1, w0, w1):
        wid = lax.axis_index("s") * _NC + lax.axis_index("c")
        base = wid * per_w
        idxs, rows = (i0, i1), (r0, r1)
        sa, sg, sw = (a0, a1), (g0, g1), (w0, w1)

        def fire_idx(b, i):
            pltpu.async_copy(idx_hbm.at[pl.ds(base + i * _CH, _CH)], idxs[b], sa[b])

        def wait_idx(b):
            pltpu.make_async_copy(idx_hbm.at[pl.ds(base, _CH)], idxs[b], sa[b]).wait()

        n = n_chunks
        fire_idx(0, 0)
        wait_idx(0)
        pltpu.async_copy(table_hbm.at[idxs[0]], rows[0], sg[0])
        fire_idx(1, 1)

        def pair(g, carry):
            for b in (0, 1):
                i = 2 * g + b
                bn = b ^ 1

                @pl.when(i + 1 < n)
                def _():
                    wait_idx(bn)

                    @pl.when(i >= 1)
                    def _():
                        pltpu.make_async_copy(
                            rows[bn], out_hbm.at[pl.ds(base, _CH)], sw[bn]).wait()

                    pltpu.async_copy(table_hbm.at[idxs[bn]], rows[bn], sg[bn])

                pltpu.make_async_copy(table_hbm.at[idxs[b]], rows[b], sg[b]).wait()

                @pl.when(i + 2 < n)
                def _():
                    fire_idx(b, i + 2)

                pltpu.async_copy(
                    rows[b], out_hbm.at[pl.ds(base + i * _CH, _CH)], sw[b])
            return carry

        lax.fori_loop(0, n // 2, pair, 0)
        pltpu.make_async_copy(rows[0], out_hbm.at[pl.ds(base, _CH)], sw[0]).wait()
        pltpu.make_async_copy(rows[1], out_hbm.at[pl.ds(base, _CH)], sw[1]).wait()

    return gk


def _sc_gather(table, idx_padded):
    return _make_gather(table.shape[0], idx_padded.shape[0])(table, idx_padded)


@functools.lru_cache(maxsize=None)
def _make_counts(Kp, Mp):
    chunks = Kp // _CH
    per_sc = chunks // 2
    per_tec = per_sc // _NS
    nz = Mp // 512
    zbound = (nz + _NS - 1) // _NS

    @functools.partial(
        pl.kernel,
        mesh=_mesh(),
        out_type=jax.ShapeDtypeStruct((2 * Mp,), jnp.float32),
        scratch_types=[
            pltpu.VMEM((_CH,), jnp.int32),
            pltpu.VMEM((_CH,), jnp.float32),
            pltpu.VMEM((512,), jnp.float32),
            pltpu.VMEM_SHARED((Mp,), jnp.float32),
        ],
    )
    def ck(idx_hbm, out_hbm, idx_v, ones_v, zbuf, acc):
        cid = lax.axis_index("c")
        sid = lax.axis_index("s")
        for i in range(32):
            zbuf[pl.ds(16 * i, 16)] = jnp.zeros((16,), jnp.float32)
        for i in range(8):
            ones_v[pl.ds(16 * i, 16)] = jnp.full((16,), 1.0, jnp.float32)
        for i in range(zbound):
            zc = sid + _NS * i

            @pl.when(zc < nz)
            def _():
                pltpu.sync_copy(zbuf, acc.at[pl.ds(zc * 512, 512)])

        plsc.subcore_barrier()

        def step(i, carry):
            c = cid * per_sc + sid * per_tec + i
            pltpu.sync_copy(idx_hbm.at[pl.ds(c * _CH, _CH)], idx_v)
            pltpu.sync_copy(ones_v, acc.at[idx_v], add=True)
            return carry

        lax.fori_loop(0, per_tec, step, 0)
        plsc.subcore_barrier()
        for i in range(zbound):
            zc = sid + _NS * i

            @pl.when(zc < nz)
            def _():
                pltpu.sync_copy(acc.at[pl.ds(zc * 512, 512)], zbuf)
                pltpu.sync_copy(zbuf, out_hbm.at[pl.ds(cid * Mp + zc * 512, 512)])

    return ck


def _sc_counts(idx_padded, Mp):
    out = _make_counts(idx_padded.shape[0], Mp)(idx_padded)
    return out.reshape(2, Mp)


@functools.lru_cache(maxsize=None)
def _make_edge_agg(EPp, HSrows, SEGrows, n_pass):
    chunks = EPp // _ECH
    per_tec = chunks // _NS
    nzr = (_R + _ECH) // _ECH
    nwr = _R // _ECH
    zbound = (nzr + _NS - 1) // _NS
    wbound = (nwr + _NS - 1) // _NS

    @functools.partial(
        pl.kernel,
        mesh=_mesh(),
        out_type=jax.ShapeDtypeStruct((n_pass * 2 * _R, _D), jnp.float32),
        scratch_types=[
            pltpu.VMEM((_ECH,), jnp.int32),
            pltpu.VMEM((_ECH,), jnp.int32),
            pltpu.VMEM((_ECH,), jnp.int32),
            pltpu.VMEM((_ECH,), jnp.int32),
            pltpu.VMEM((_ECH,), jnp.float32),
            pltpu.VMEM((_ECH,), jnp.float32),
            pltpu.VMEM((_ECH,), jnp.int32),
            pltpu.VMEM((_ECH,), jnp.int32),
            pltpu.VMEM((_ECH,), jnp.int32),
            pltpu.VMEM((_ECH, _D), jnp.float32),
            pltpu.VMEM((_ECH, _D), jnp.float32),
            pltpu.SemaphoreType.DMA,
            pltpu.SemaphoreType.DMA,
            pltpu.SemaphoreType.DMA,
            pltpu.SemaphoreType.DMA,
            pltpu.VMEM_SHARED((_R + _ECH, _D), jnp.float32),
        ],
    )
    def ek(hs_hbm, src_hbm, dst_hbm, seg_hbm, dinv_hbm, out_hbm,
           s0, s1, d0, d1, v0, v1, e0, e1, lidx_v, r0, r1,
           a0, a1, g0, g1, acc):
        cid = lax.axis_index("c")
        sid = lax.axis_index("s")
        srcs, dsts, dinvds, segds = (s0, s1), (d0, d1), (v0, v1), (e0, e1)
        rows = (r0, r1)
        sa, sg = (a0, a1), (g0, g1)
        tbase = sid * per_tec
        n = per_tec

        def fire_idx(b, i):
            off = (tbase + i) * _ECH
            pltpu.async_copy(src_hbm.at[pl.ds(off, _ECH)], srcs[b], sa[b])
            pltpu.async_copy(dst_hbm.at[pl.ds(off, _ECH)], dsts[b], sa[b])

        def wait_idx(b):
            pltpu.make_async_copy(src_hbm.at[pl.ds(0, _ECH)], srcs[b], sa[b]).wait()
            pltpu.make_async_copy(dst_hbm.at[pl.ds(0, _ECH)], dsts[b], sa[b]).wait()

        def fire_gat(b):
            pltpu.async_copy(dinv_hbm.at[dsts[b]], dinvds[b], sg[b])
            pltpu.async_copy(seg_hbm.at[dsts[b]], segds[b], sg[b])
            pltpu.async_copy(hs_hbm.at[srcs[b]], rows[b], sg[b])

        def wait_gat(b):
            pltpu.make_async_copy(dinv_hbm.at[dsts[b]], dinvds[b], sg[b]).wait()
            pltpu.make_async_copy(seg_hbm.at[dsts[b]], segds[b], sg[b]).wait()
            pltpu.make_async_copy(hs_hbm.at[srcs[b]], rows[b], sg[b]).wait()

        for p in range(n_pass):
            base = (2 * p + cid) * _R

            def zrow(j, carry):
                for q in range(8):
                    r0[j, pl.ds(16 * q, 16)] = jnp.zeros((16,), jnp.float32)
                return carry

            lax.fori_loop(0, _ECH, zrow, 0)
            for i in range(zbound):
                zc = sid + _NS * i

                @pl.when(zc < nzr)
                def _():
                    pltpu.sync_copy(r0, acc.at[pl.ds(zc * _ECH, _ECH)])

            plsc.subcore_barrier()

            fire_idx(0, 0)
            wait_idx(0)
            fire_gat(0)
            fire_idx(1, 1)

            def pair(g, carry):
                for b in (0, 1):
                    i = 2 * g + b
                    bn = b ^ 1

                    @pl.when(i + 1 < n)
                    def _():
                        wait_idx(bn)
                        fire_gat(bn)

                    wait_gat(b)

                    @pl.when(i + 2 < n)
                    def _():
                        fire_idx(b, i + 2)

                    for q in range(_ECH // 16):
                        sl = pl.ds(16 * q, 16)
                        v = segds[b][sl] - base
                        m = (v >= 0) & (v < _R)
                        lidx_v[sl] = jnp.where(m, v, _R)

                    def sgrp(t, c2):
                        w16 = dinvds[b][pl.ds(16 * t, 16)]
                        for l in range(16):
                            s = w16[l]
                            j = 16 * t + l
                            for q in range(8):
                                sl = pl.ds(16 * q, 16)
                                rows[b][j, sl] = rows[b][j, sl] * s
                        return c2

                    lax.fori_loop(0, _ECH // 16, sgrp, 0)
                    pltpu.sync_copy(rows[b], acc.at[lidx_v], add=True)
                return carry

            lax.fori_loop(0, n // 2, pair, 0)
            plsc.subcore_barrier()
            for i in range(wbound):
                wc = sid + _NS * i

                @pl.when(wc < nwr)
                def _():
                    pltpu.sync_copy(acc.at[pl.ds(wc * _ECH, _ECH)], r0)
                    pltpu.sync_copy(r0, out_hbm.at[pl.ds(base + wc * _ECH, _ECH)])

            plsc.subcore_barrier()

    return ek


def _sc_edge_agg(hs, src, dst, seg, dinv, n_pass):
    k = _make_edge_agg(src.shape[0], hs.shape[0], seg.shape[0], n_pass)
    return k(hs, src, dst, seg, dinv)


def _tc_matmul(g, W, dinv_col):
    Kp = g.shape[0]
    blk = 512

    def body(g_ref, w_ref, d_ref, o_ref):
        o_ref[...] = jnp.dot(g_ref[...], w_ref[...],
                             preferred_element_type=jnp.float32) * d_ref[...]

    return pl.pallas_call(
        body,
        grid=(Kp // blk,),
        in_specs=[
            pl.BlockSpec((blk, _D), lambda i: (i, 0)),
            pl.BlockSpec((_D, _D), lambda i: (0, 0)),
            pl.BlockSpec((blk, 1), lambda i: (i, 0)),
        ],
        out_specs=pl.BlockSpec((blk, _D), lambda i: (i, 0)),
        out_shape=jax.ShapeDtypeStruct((Kp, _D), jnp.float32),
    )(g, W, dinv_col)


def _tc_dinv(deg_parts):
    rows = deg_parts.shape[1] // _D

    def body(d_ref, o_ref):
        d = d_ref[0] + d_ref[1]
        o_ref[...] = lax.rsqrt(jnp.maximum(d, 1.0))

    return pl.pallas_call(
        body,
        out_shape=jax.ShapeDtypeStruct((rows, _D), jnp.float32),
    )(deg_parts.reshape(2, rows, _D))


def _tc_post(ssum, cparts, bias, ln_g, ln_b):
    Mp = ssum.shape[0]
    blk = 512
    c0 = cparts[0].reshape(Mp, 1)
    c1 = cparts[1].reshape(Mp, 1)

    def body(s_ref, c0_ref, c1_ref, b_ref, g_ref, lb_ref, o_ref):
        c = jnp.maximum(c0_ref[...] + c1_ref[...], 1.0)
        ex = s_ref[...] * (1.0 / c) + b_ref[...]
        m = jnp.mean(ex, axis=-1, keepdims=True)
        v = jnp.mean((ex - m) ** 2, axis=-1, keepdims=True)
        y = g_ref[...] * (ex - m) / jnp.sqrt(v + 1e-5) + lb_ref[...]
        o_ref[...] = jnp.where(y > 0, y, jnp.exp(jnp.minimum(y, 0.0)) - 1.0)

    return pl.pallas_call(
        body,
        grid=(Mp // blk,),
        in_specs=[
            pl.BlockSpec((blk, _D), lambda i: (i, 0)),
            pl.BlockSpec((blk, 1), lambda i: (i, 0)),
            pl.BlockSpec((blk, 1), lambda i: (i, 0)),
            pl.BlockSpec((1, _D), lambda i: (0, 0)),
            pl.BlockSpec((1, _D), lambda i: (0, 0)),
            pl.BlockSpec((1, _D), lambda i: (0, 0)),
        ],
        out_specs=pl.BlockSpec((blk, _D), lambda i: (i, 0)),
        out_shape=jax.ShapeDtypeStruct((Mp, _D), jnp.float32),
    )(ssum, c0, c1, bias.reshape(1, _D), ln_g.reshape(1, _D), ln_b.reshape(1, _D))


def _tc_final(nx1, nx2, n2g3, l1a, l1b2, lb1, w2p, b2p):
    blk = 512
    ngrid = _N_PAD // blk

    def body(x1_ref, x2_ref, id_ref, a_ref, b_ref, lb_ref, w2_ref, b2_ref,
             o_ref, acc1, acc2, cnt):
        i = pl.program_id(0)

        @pl.when(i == 0)
        def _():
            acc1[...] = jnp.zeros_like(acc1)
            acc2[...] = jnp.zeros_like(acc2)
            cnt[...] = jnp.zeros_like(cnt)

        ids = id_ref[0, 0, :]
        oh = (lax.broadcasted_iota(jnp.int32, (_G, blk), 0)
              == ids[None, :]).astype(jnp.float32)
        acc1[...] += jnp.dot(oh, x1_ref[...], preferred_element_type=jnp.float32)
        acc2[...] += jnp.dot(oh, x2_ref[...], preferred_element_type=jnp.float32)
        cnt[...] += jnp.sum(oh, axis=1, keepdims=True)

        @pl.when(i == ngrid - 1)
        def _():
            ci = 1.0 / jnp.maximum(cnt[...], 1.0)
            g1 = acc1[...] * ci
            g2 = acc2[...] * ci
            h1 = jnp.maximum(
                jnp.dot(g1, a_ref[...], preferred_element_type=jnp.float32)
                + jnp.dot(g2, b_ref[...], preferred_element_type=jnp.float32)
                + lb_ref[...], 0.0)
            lg = jnp.dot(h1, w2_ref[...],
                         preferred_element_type=jnp.float32) + b2_ref[...]
            col = lax.broadcasted_iota(jnp.int32, (_G, _D), 1)
            lg = jnp.where(col < _C, lg, -1e30)
            m = jnp.max(lg, axis=-1, keepdims=True)
            lse = jnp.log(jnp.sum(jnp.exp(lg - m), axis=-1, keepdims=True)) + m
            o_ref[...] = lg - lse

    return pl.pallas_call(
        body,
        grid=(ngrid,),
        in_specs=[
            pl.BlockSpec((blk, _D), lambda i: (i, 0)),
            pl.BlockSpec((blk, _D), lambda i: (i, 0)),
            pl.BlockSpec((1, 1, blk), lambda i: (i, 0, 0)),
            pl.BlockSpec((_D, _D), lambda i: (0, 0)),
            pl.BlockSpec((_D, _D), lambda i: (0, 0)),
            pl.BlockSpec((1, _D), lambda i: (0, 0)),
            pl.BlockSpec((_D, _D), lambda i: (0, 0)),
            pl.BlockSpec((1, _D), lambda i: (0, 0)),
        ],
        out_specs=pl.BlockSpec((_G, _D), lambda i: (0, 0)),
        out_shape=jax.ShapeDtypeStruct((_G, _D), jnp.float32),
        scratch_shapes=[
            pltpu.VMEM((_G, _D), jnp.float32),
            pltpu.VMEM((_G, _D), jnp.float32),
            pltpu.VMEM((_G, 1), jnp.float32),
        ],
    )(nx1, nx2, n2g3, l1a, l1b2, lb1, w2p, b2p)


def kernel(x, eg_orig_node_idx, eg_node_to_hedge, eg_edge_index, ng_orig_edge_idx, ng_hedge_to_node, ng_edge_index, node_to_graph, W_n2e_0, b_n2e_0, W_e2n_0, b_e2n_0, W_n2e_1, b_n2e_1, W_e2n_1, b_e2n_1, ln_e_g, ln_e_b, ln_n_g, ln_n_b, lin1_W, lin1_b, lin2_W, lin2_b):
    i32 = jnp.int32
    padP = _P_PAD - _P
    padE = _EP_PAD - (_E + _P)
    loop = jnp.arange(_P, dtype=i32)

    eg_oni = jnp.concatenate([eg_orig_node_idx, jnp.zeros((padP,), i32)])
    ng_oei = jnp.concatenate([ng_orig_edge_idx, jnp.zeros((padP,), i32)])
    seg_eg = jnp.concatenate([eg_node_to_hedge, jnp.full((padP,), _BIG, i32)])
    seg_ng = jnp.concatenate([ng_hedge_to_node, jnp.full((padP,), _BIG, i32)])
    cidx_eg = jnp.concatenate([eg_node_to_hedge, jnp.full((padP,), _H_PAD - 8, i32)])
    cidx_ng = jnp.concatenate([ng_hedge_to_node, jnp.full((padP,), _N_PAD - 8, i32)])
    src_eg = jnp.concatenate([eg_edge_index[0], loop, jnp.zeros((padE,), i32)])
    dst_eg = jnp.concatenate([eg_edge_index[1], loop, jnp.full((padE,), _P, i32)])
    src_ng = jnp.concatenate([ng_edge_index[0], loop, jnp.zeros((padE,), i32)])
    dst_ng = jnp.concatenate([ng_edge_index[1], loop, jnp.full((padE,), _P, i32)])
    n2g3 = jnp.concatenate([node_to_graph, jnp.full((_N_PAD - _N,), _G, i32)]
                           ).reshape(_N_PAD // 512, 1, 512)

    deg_eg = _sc_counts(dst_eg, _P_PAD)
    deg_ng = _sc_counts(dst_ng, _P_PAD)
    c_eg = _sc_counts(cidx_eg, _H_PAD)
    c_ng = _sc_counts(cidx_ng, _N_PAD)

    dinv_eg = _tc_dinv(deg_eg).reshape(_P_PAD)
    dinv_ng = _tc_dinv(deg_ng).reshape(_P_PAD)
    dinv_eg_col = dinv_eg.reshape(_P_PAD, 1)
    dinv_ng_col = dinv_ng.reshape(_P_PAD, 1)

    def layer(nx_full, Wn, bn, We, be):
        g1 = _sc_gather(nx_full, eg_oni)
        hs1 = _tc_matmul(g1, Wn, dinv_eg_col)
        s1 = _sc_edge_agg(hs1, src_eg, dst_eg, seg_eg, dinv_eg, 1)
        ex = _tc_post(s1, c_eg, bn, ln_e_g, ln_e_b)
        g2 = _sc_gather(ex, ng_oei)
        hs2 = _tc_matmul(g2, We, dinv_ng_col)
        s2 = _sc_edge_agg(hs2, src_ng, dst_ng, seg_ng, dinv_ng, 2)
        return _tc_post(s2, c_ng, be, ln_n_g, ln_n_b)

    nx1 = layer(x, W_n2e_0, b_n2e_0, W_e2n_0, b_e2n_0)
    nx2 = layer(nx1, W_n2e_1, b_n2e_1, W_e2n_1, b_e2n_1)

    l1a = lin1_W[:_D]
    l1b2 = lin1_W[_D:]
    w2p = jnp.zeros((_D, _D), jnp.float32).at[:, :_C].set(lin2_W)
    b2p = jnp.zeros((1, _D), jnp.float32).at[0, :_C].set(lin2_b)
    out = _tc_final(nx1, nx2, n2g3, l1a, l1b2, lin1_b.reshape(1, _D), w2p, b2p)
    return out[:, :_C]


# speedup vs baseline: 4.6355x; 1.0122x over previous
import functools

import jax
import jax.numpy as jnp
from jax import lax
from jax.experimental import pallas as pl
from jax.experimental.pallas import tpu as pltpu
from jax.experimental.pallas import tpu_sc as plsc

_NC = 2
_NS = 16
_NW = _NC * _NS
_CH = 128
_ECH = 64
_D = 128

_N = 50000
_H = 25000
_P = 400000
_E = 400000
_G = 64
_C = 10

_P_PAD = 401408
_H_PAD = 25088
_N_PAD = 50176
_EP_PAD = 802816
_R = 12544

_BIG = 1 << 30


def _pad_len(n, mult):
    return ((n + mult - 1) // mult) * mult


def _mesh():
    return plsc.VectorSubcoreMesh(core_axis_name="c", subcore_axis_name="s",
                                  num_cores=_NC, num_subcores=_NS)


@functools.lru_cache(maxsize=None)
def _make_gather(M, Kp):
    per_w = Kp // _NW
    n_chunks = per_w // _CH

    @functools.partial(
        pl.kernel,
        mesh=_mesh(),
        out_type=jax.ShapeDtypeStruct((Kp, _D), jnp.float32),
        scratch_types=[
            pltpu.VMEM((_CH,), jnp.int32),
            pltpu.VMEM((_CH,), jnp.int32),
            pltpu.VMEM((_CH, _D), jnp.float32),
            pltpu.VMEM((_CH, _D), jnp.float32),
            pltpu.SemaphoreType.DMA,
            pltpu.SemaphoreType.DMA,
            pltpu.SemaphoreType.DMA,
            pltpu.SemaphoreType.DMA,
            pltpu.SemaphoreType.DMA,
            pltpu.SemaphoreType.DMA,
        ],
    )
    def gk(table_hbm, idx_hbm, out_hbm, i0, i1, r0, r1, a0, a1, g0, g1, w0, w1):
        wid = lax.axis_index("s") * _NC + lax.axis_index("c")
        base = wid * per_w
        idxs, rows = (i0, i1), (r0, r1)
        sa, sg, sw = (a0, a1), (g0, g1), (w0, w1)

        def fire_idx(b, i):
            pltpu.async_copy(idx_hbm.at[pl.ds(base + i * _CH, _CH)], idxs[b], sa[b])

        def wait_idx(b):
            pltpu.make_async_copy(idx_hbm.at[pl.ds(base, _CH)], idxs[b], sa[b]).wait()

        n = n_chunks
        fire_idx(0, 0)
        wait_idx(0)
        pltpu.async_copy(table_hbm.at[idxs[0]], rows[0], sg[0])
        fire_idx(1, 1)

        def pair(g, carry):
            for b in (0, 1):
                i = 2 * g + b
                bn = b ^ 1

                @pl.when(i + 1 < n)
                def _():
                    wait_idx(bn)

                    @pl.when(i >= 1)
                    def _():
                        pltpu.make_async_copy(
                            rows[bn], out_hbm.at[pl.ds(base, _CH)], sw[bn]).wait()

                    pltpu.async_copy(table_hbm.at[idxs[bn]], rows[bn], sg[bn])

                pltpu.make_async_copy(table_hbm.at[idxs[b]], rows[b], sg[b]).wait()

                @pl.when(i + 2 < n)
                def _():
                    fire_idx(b, i + 2)

                pltpu.async_copy(
                    rows[b], out_hbm.at[pl.ds(base + i * _CH, _CH)], sw[b])
            return carry

        lax.fori_loop(0, n // 2, pair, 0)
        pltpu.make_async_copy(rows[0], out_hbm.at[pl.ds(base, _CH)], sw[0]).wait()
        pltpu.make_async_copy(rows[1], out_hbm.at[pl.ds(base, _CH)], sw[1]).wait()

    return gk


def _sc_gather(table, idx_padded):
    return _make_gather(table.shape[0], idx_padded.shape[0])(table, idx_padded)


@functools.lru_cache(maxsize=None)
def _make_counts(Kp, Mp):
    chunks = Kp // _CH
    per_sc = chunks // 2
    per_tec = per_sc // _NS
    nz = Mp // 512
    zbound = (nz + _NS - 1) // _NS

    @functools.partial(
        pl.kernel,
        mesh=_mesh(),
        out_type=jax.ShapeDtypeStruct((2 * Mp,), jnp.float32),
        scratch_types=[
            pltpu.VMEM((_CH,), jnp.int32),
            pltpu.VMEM((_CH,), jnp.int32),
            pltpu.VMEM((_CH,), jnp.float32),
            pltpu.VMEM((512,), jnp.float32),
            pltpu.SemaphoreType.DMA,
            pltpu.SemaphoreType.DMA,
            pltpu.VMEM_SHARED((Mp,), jnp.float32),
        ],
    )
    def ck(idx_hbm, out_hbm, x0, x1, ones_v, zbuf, a0, a1, acc):
        cid = lax.axis_index("c")
        sid = lax.axis_index("s")
        idxs, sa = (x0, x1), (a0, a1)
        for i in range(32):
            zbuf[pl.ds(16 * i, 16)] = jnp.zeros((16,), jnp.float32)
        for i in range(8):
            ones_v[pl.ds(16 * i, 16)] = jnp.full((16,), 1.0, jnp.float32)
        for i in range(zbound):
            zc = sid + _NS * i

            @pl.when(zc < nz)
            def _():
                pltpu.sync_copy(zbuf, acc.at[pl.ds(zc * 512, 512)])

        plsc.subcore_barrier()

        cbase = cid * per_sc + sid * per_tec
        n = per_tec

        def fire(b, i):
            pltpu.async_copy(idx_hbm.at[pl.ds((cbase + i) * _CH, _CH)],
                             idxs[b], sa[b])

        def wait(b):
            pltpu.make_async_copy(idx_hbm.at[pl.ds(0, _CH)], idxs[b], sa[b]).wait()

        fire(0, 0)

        def pair(g, carry):
            for b in (0, 1):
                i = 2 * g + b

                @pl.when(i + 1 < n)
                def _():
                    fire(b ^ 1, i + 1)

                wait(b)
                pltpu.sync_copy(ones_v, acc.at[idxs[b]], add=True)
            return carry

        lax.fori_loop(0, n // 2, pair, 0)
        plsc.subcore_barrier()
        for i in range(zbound):
            zc = sid + _NS * i

            @pl.when(zc < nz)
            def _():
                pltpu.sync_copy(acc.at[pl.ds(zc * 512, 512)], zbuf)
                pltpu.sync_copy(zbuf, out_hbm.at[pl.ds(cid * Mp + zc * 512, 512)])

    return ck


def _sc_counts(idx_padded, Mp):
    out = _make_counts(idx_padded.shape[0], Mp)(idx_padded)
    return out.reshape(2, Mp)


@functools.lru_cache(maxsize=None)
def _make_edge_agg(EPp, HSrows, SEGrows, n_pass):
    chunks = EPp // _ECH
    per_tec = chunks // _NS
    nzr = (_R + _ECH) // _ECH
    nwr = _R // _ECH
    zbound = (nzr + _NS - 1) // _NS
    wbound = (nwr + _NS - 1) // _NS

    @functools.partial(
        pl.kernel,
        mesh=_mesh(),
        out_type=jax.ShapeDtypeStruct((n_pass * 2 * _R, _D), jnp.float32),
        scratch_types=[
            pltpu.VMEM((_ECH,), jnp.int32),
            pltpu.VMEM((_ECH,), jnp.int32),
            pltpu.VMEM((_ECH,), jnp.int32),
            pltpu.VMEM((_ECH,), jnp.int32),
            pltpu.VMEM((_ECH,), jnp.float32),
            pltpu.VMEM((_ECH,), jnp.float32),
            pltpu.VMEM((_ECH,), jnp.int32),
            pltpu.VMEM((_ECH,), jnp.int32),
            pltpu.VMEM((_ECH,), jnp.int32),
            pltpu.VMEM((_ECH, _D), jnp.float32),
            pltpu.VMEM((_ECH, _D), jnp.float32),
            pltpu.SemaphoreType.DMA,
            pltpu.SemaphoreType.DMA,
            pltpu.SemaphoreType.DMA,
            pltpu.SemaphoreType.DMA,
            pltpu.VMEM_SHARED((_R + _ECH, _D), jnp.float32),
        ],
    )
    def ek(hs_hbm, src_hbm, dst_hbm, seg_hbm, dinv_hbm, out_hbm,
           s0, s1, d0, d1, v0, v1, e0, e1, lidx_v, r0, r1,
           a0, a1, g0, g1, acc):
        cid = lax.axis_index("c")
        sid = lax.axis_index("s")
        srcs, dsts, dinvds, segds = (s0, s1), (d0, d1), (v0, v1), (e0, e1)
        rows = (r0, r1)
        sa, sg = (a0, a1), (g0, g1)
        tbase = sid * per_tec
        n = per_tec

        def fire_idx(b, i):
            off = (tbase + i) * _ECH
            pltpu.async_copy(src_hbm.at[pl.ds(off, _ECH)], srcs[b], sa[b])
            pltpu.async_copy(dst_hbm.at[pl.ds(off, _ECH)], dsts[b], sa[b])

        def wait_idx(b):
            pltpu.make_async_copy(src_hbm.at[pl.ds(0, _ECH)], srcs[b], sa[b]).wait()
            pltpu.make_async_copy(dst_hbm.at[pl.ds(0, _ECH)], dsts[b], sa[b]).wait()

        def fire_gat(b):
            pltpu.async_copy(dinv_hbm.at[dsts[b]], dinvds[b], sg[b])
            pltpu.async_copy(seg_hbm.at[dsts[b]], segds[b], sg[b])
            pltpu.async_copy(hs_hbm.at[srcs[b]], rows[b], sg[b])

        def wait_gat(b):
            pltpu.make_async_copy(dinv_hbm.at[dsts[b]], dinvds[b], sg[b]).wait()
            pltpu.make_async_copy(seg_hbm.at[dsts[b]], segds[b], sg[b]).wait()
            pltpu.make_async_copy(hs_hbm.at[srcs[b]], rows[b], sg[b]).wait()

        for p in range(n_pass):
            base = (2 * p + cid) * _R

            def zrow(j, carry):
                for q in range(8):
                    r0[j, pl.ds(16 * q, 16)] = jnp.zeros((16,), jnp.float32)
                return carry

            lax.fori_loop(0, _ECH, zrow, 0)
            for i in range(zbound):
                zc = sid + _NS * i

                @pl.when(zc < nzr)
                def _():
                    pltpu.sync_copy(r0, acc.at[pl.ds(zc * _ECH, _ECH)])

            plsc.subcore_barrier()

            fire_idx(0, 0)
            wait_idx(0)
            fire_gat(0)
            fire_idx(1, 1)

            def pair(g, carry):
                for b in (0, 1):
                    i = 2 * g + b
                    bn = b ^ 1

                    @pl.when(i + 1 < n)
                    def _():
                        wait_idx(bn)
                        fire_gat(bn)

                    wait_gat(b)

                    @pl.when(i + 2 < n)
                    def _():
                        fire_idx(b, i + 2)

                    for q in range(_ECH // 16):
                        sl = pl.ds(16 * q, 16)
                        v = segds[b][sl] - base
                        m = (v >= 0) & (v < _R)
                        lidx_v[sl] = jnp.where(m, v, _R)

                    def sgrp(t, c2):
                        w16 = dinvds[b][pl.ds(16 * t, 16)]
                        for l in range(16):
                            s = w16[l]
                            j = 16 * t + l
                            for q in range(8):
                                sl = pl.ds(16 * q, 16)
                                rows[b][j, sl] = rows[b][j, sl] * s
                        return c2

                    lax.fori_loop(0, _ECH // 16, sgrp, 0)
                    pltpu.sync_copy(rows[b], acc.at[lidx_v], add=True)
                return carry

            lax.fori_loop(0, n // 2, pair, 0)
            plsc.subcore_barrier()
            for i in range(wbound):
                wc = sid + _NS * i

                @pl.when(wc < nwr)
                def _():
                    pltpu.sync_copy(acc.at[pl.ds(wc * _ECH, _ECH)], r0)
                    pltpu.sync_copy(r0, out_hbm.at[pl.ds(base + wc * _ECH, _ECH)])

            plsc.subcore_barrier()

    return ek


def _sc_edge_agg(hs, src, dst, seg, dinv, n_pass):
    k = _make_edge_agg(src.shape[0], hs.shape[0], seg.shape[0], n_pass)
    return k(hs, src, dst, seg, dinv)


def _tc_matmul(g, W, dinv_col):
    Kp = g.shape[0]
    blk = 512

    def body(g_ref, w_ref, d_ref, o_ref):
        o_ref[...] = jnp.dot(g_ref[...], w_ref[...],
                             preferred_element_type=jnp.float32) * d_ref[...]

    return pl.pallas_call(
        body,
        grid=(Kp // blk,),
        in_specs=[
            pl.BlockSpec((blk, _D), lambda i: (i, 0)),
            pl.BlockSpec((_D, _D), lambda i: (0, 0)),
            pl.BlockSpec((blk, 1), lambda i: (i, 0)),
        ],
        out_specs=pl.BlockSpec((blk, _D), lambda i: (i, 0)),
        out_shape=jax.ShapeDtypeStruct((Kp, _D), jnp.float32),
    )(g, W, dinv_col)


def _tc_dinv(deg_parts):
    rows = deg_parts.shape[1] // _D

    def body(d_ref, o_ref):
        d = d_ref[0] + d_ref[1]
        o_ref[...] = lax.rsqrt(jnp.maximum(d, 1.0))

    return pl.pallas_call(
        body,
        out_shape=jax.ShapeDtypeStruct((rows, _D), jnp.float32),
    )(deg_parts.reshape(2, rows, _D))


def _tc_post(ssum, cparts, bias, ln_g, ln_b):
    Mp = ssum.shape[0]
    blk = 512
    c0 = cparts[0].reshape(Mp, 1)
    c1 = cparts[1].reshape(Mp, 1)

    def body(s_ref, c0_ref, c1_ref, b_ref, g_ref, lb_ref, o_ref):
        c = jnp.maximum(c0_ref[...] + c1_ref[...], 1.0)
        ex = s_ref[...] * (1.0 / c) + b_ref[...]
        m = jnp.mean(ex, axis=-1, keepdims=True)
        v = jnp.mean((ex - m) ** 2, axis=-1, keepdims=True)
        y = g_ref[...] * (ex - m) / jnp.sqrt(v + 1e-5) + lb_ref[...]
        o_ref[...] = jnp.where(y > 0, y, jnp.exp(jnp.minimum(y, 0.0)) - 1.0)

    return pl.pallas_call(
        body,
        grid=(Mp // blk,),
        in_specs=[
            pl.BlockSpec((blk, _D), lambda i: (i, 0)),
            pl.BlockSpec((blk, 1), lambda i: (i, 0)),
            pl.BlockSpec((blk, 1), lambda i: (i, 0)),
            pl.BlockSpec((1, _D), lambda i: (0, 0)),
            pl.BlockSpec((1, _D), lambda i: (0, 0)),
            pl.BlockSpec((1, _D), lambda i: (0, 0)),
        ],
        out_specs=pl.BlockSpec((blk, _D), lambda i: (i, 0)),
        out_shape=jax.ShapeDtypeStruct((Mp, _D), jnp.float32),
    )(ssum, c0, c1, bias.reshape(1, _D), ln_g.reshape(1, _D), ln_b.reshape(1, _D))


def _tc_final(nx1, nx2, n2g3, l1a, l1b2, lb1, w2p, b2p):
    blk = 512
    ngrid = _N_PAD // blk

    def body(x1_ref, x2_ref, id_ref, a_ref, b_ref, lb_ref, w2_ref, b2_ref,
             o_ref, acc1, acc2, cnt):
        i = pl.program_id(0)

        @pl.when(i == 0)
        def _():
            acc1[...] = jnp.zeros_like(acc1)
            acc2[...] = jnp.zeros_like(acc2)
            cnt[...] = jnp.zeros_like(cnt)

        ids = id_ref[0, 0, :]
        oh = (lax.broadcasted_iota(jnp.int32, (_G, blk), 0)
              == ids[None, :]).astype(jnp.float32)
        acc1[...] += jnp.dot(oh, x1_ref[...], preferred_element_type=jnp.float32)
        acc2[...] += jnp.dot(oh, x2_ref[...], preferred_element_type=jnp.float32)
        cnt[...] += jnp.sum(oh, axis=1, keepdims=True)

        @pl.when(i == ngrid - 1)
        def _():
            ci = 1.0 / jnp.maximum(cnt[...], 1.0)
            g1 = acc1[...] * ci
            g2 = acc2[...] * ci
            h1 = jnp.maximum(
                jnp.dot(g1, a_ref[...], preferred_element_type=jnp.float32)
                + jnp.dot(g2, b_ref[...], preferred_element_type=jnp.float32)
                + lb_ref[...], 0.0)
            lg = jnp.dot(h1, w2_ref[...],
                         preferred_element_type=jnp.float32) + b2_ref[...]
            col = lax.broadcasted_iota(jnp.int32, (_G, _D), 1)
            lg = jnp.where(col < _C, lg, -1e30)
            m = jnp.max(lg, axis=-1, keepdims=True)
            lse = jnp.log(jnp.sum(jnp.exp(lg - m), axis=-1, keepdims=True)) + m
            o_ref[...] = lg - lse

    return pl.pallas_call(
        body,
        grid=(ngrid,),
        in_specs=[
            pl.BlockSpec((blk, _D), lambda i: (i, 0)),
            pl.BlockSpec((blk, _D), lambda i: (i, 0)),
            pl.BlockSpec((1, 1, blk), lambda i: (i, 0, 0)),
            pl.BlockSpec((_D, _D), lambda i: (0, 0)),
            pl.BlockSpec((_D, _D), lambda i: (0, 0)),
            pl.BlockSpec((1, _D), lambda i: (0, 0)),
            pl.BlockSpec((_D, _D), lambda i: (0, 0)),
            pl.BlockSpec((1, _D), lambda i: (0, 0)),
        ],
        out_specs=pl.BlockSpec((_G, _D), lambda i: (0, 0)),
        out_shape=jax.ShapeDtypeStruct((_G, _D), jnp.float32),
        scratch_shapes=[
            pltpu.VMEM((_G, _D), jnp.float32),
            pltpu.VMEM((_G, _D), jnp.float32),
            pltpu.VMEM((_G, 1), jnp.float32),
        ],
    )(nx1, nx2, n2g3, l1a, l1b2, lb1, w2p, b2p)


def kernel(x, eg_orig_node_idx, eg_node_to_hedge, eg_edge_index, ng_orig_edge_idx, ng_hedge_to_node, ng_edge_index, node_to_graph, W_n2e_0, b_n2e_0, W_e2n_0, b_e2n_0, W_n2e_1, b_n2e_1, W_e2n_1, b_e2n_1, ln_e_g, ln_e_b, ln_n_g, ln_n_b, lin1_W, lin1_b, lin2_W, lin2_b):
    i32 = jnp.int32
    padP = _P_PAD - _P
    padE = _EP_PAD - (_E + _P)
    loop = jnp.arange(_P, dtype=i32)

    eg_oni = jnp.concatenate([eg_orig_node_idx, jnp.zeros((padP,), i32)])
    ng_oei = jnp.concatenate([ng_orig_edge_idx, jnp.zeros((padP,), i32)])
    seg_eg = jnp.concatenate([eg_node_to_hedge, jnp.full((padP,), _BIG, i32)])
    seg_ng = jnp.concatenate([ng_hedge_to_node, jnp.full((padP,), _BIG, i32)])
    cidx_eg = jnp.concatenate([eg_node_to_hedge, jnp.full((padP,), _H_PAD - 8, i32)])
    cidx_ng = jnp.concatenate([ng_hedge_to_node, jnp.full((padP,), _N_PAD - 8, i32)])
    src_eg = jnp.concatenate([eg_edge_index[0], loop, jnp.zeros((padE,), i32)])
    dst_eg = jnp.concatenate([eg_edge_index[1], loop, jnp.full((padE,), _P, i32)])
    src_ng = jnp.concatenate([ng_edge_index[0], loop, jnp.zeros((padE,), i32)])
    dst_ng = jnp.concatenate([ng_edge_index[1], loop, jnp.full((padE,), _P, i32)])
    n2g3 = jnp.concatenate([node_to_graph, jnp.full((_N_PAD - _N,), _G, i32)]
                           ).reshape(_N_PAD // 512, 1, 512)

    deg_eg = _sc_counts(dst_eg, _P_PAD)
    deg_ng = _sc_counts(dst_ng, _P_PAD)
    c_eg = _sc_counts(cidx_eg, _H_PAD)
    c_ng = _sc_counts(cidx_ng, _N_PAD)

    dinv_eg = _tc_dinv(deg_eg).reshape(_P_PAD)
    dinv_ng = _tc_dinv(deg_ng).reshape(_P_PAD)
    dinv_eg_col = dinv_eg.reshape(_P_PAD, 1)
    dinv_ng_col = dinv_ng.reshape(_P_PAD, 1)

    def layer(nx_full, Wn, bn, We, be):
        g1 = _sc_gather(nx_full, eg_oni)
        hs1 = _tc_matmul(g1, Wn, dinv_eg_col)
        s1 = _sc_edge_agg(hs1, src_eg, dst_eg, seg_eg, dinv_eg, 1)
        ex = _tc_post(s1, c_eg, bn, ln_e_g, ln_e_b)
        g2 = _sc_gather(ex, ng_oei)
        hs2 = _tc_matmul(g2, We, dinv_ng_col)
        s2 = _sc_edge_agg(hs2, src_ng, dst_ng, seg_ng, dinv_ng, 2)
        return _tc_post(s2, c_ng, be, ln_n_g, ln_n_b)

    nx1 = layer(x, W_n2e_0, b_n2e_0, W_e2n_0, b_e2n_0)
    nx2 = layer(nx1, W_n2e_1, b_n2e_1, W_e2n_1, b_e2n_1)

    l1a = lin1_W[:_D]
    l1b2 = lin1_W[_D:]
    w2p = jnp.zeros((_D, _D), jnp.float32).at[:, :_C].set(lin2_W)
    b2p = jnp.zeros((1, _D), jnp.float32).at[0, :_C].set(lin2_b)
    out = _tc_final(nx1, nx2, n2g3, l1a, l1b2, lin1_b.reshape(1, _D), w2p, b2p)
    return out[:, :_C]
